# Initial kernel scaffold; baseline (speedup 1.0000x reference)
import jax
import jax.numpy as jnp
from jax.experimental import pallas as pl


def kernel(x, A, W, b):
    raise NotImplementedError("write your pallas kernel here")


# speedup vs baseline: 1.3127x; 1.3127x over previous
import jax
import jax.numpy as jnp
from jax import lax
from jax.experimental import pallas as pl
from jax.experimental.pallas import tpu as pltpu

_RB = 512


def _gcn_body(a_ref, x_ref, w_ref, b_ref, o_ref, a8_ref, d_ref, y_ref, ybf_ref):
    k = pl.program_id(0)
    nstripes = a8_ref.shape[0]

    @pl.when(k < nstripes)
    def _phase1():
        a = a_ref[...]
        a8_ref[pl.ds(k, 1), :, :] = a.astype(jnp.int8)[None]
        d_ref[pl.ds(k * _RB, _RB), :] = jnp.sum(a, axis=1, keepdims=True) + 1.0

    @pl.when(k == nstripes)
    def _prep():
        d_all = lax.rsqrt(d_ref[...])
        d_ref[...] = d_all
        xw = lax.dot_general(
            x_ref[...], w_ref[...],
            dimension_numbers=(((1,), (1,)), ((), ())),
            preferred_element_type=jnp.float32,
        )
        y = d_all * xw
        y_ref[...] = y
        ybf_ref[...] = y.astype(jnp.bfloat16)

    @pl.when(k >= nstripes)
    def _phase2():
        i = k - nstripes
        a_bf = a8_ref[pl.ds(i, 1), :, :][0].astype(jnp.bfloat16)
        z = lax.dot_general(
            a_bf, ybf_ref[...],
            dimension_numbers=(((1,), (0,)), ((), ())),
            preferred_element_type=jnp.float32,
        )
        d_blk = d_ref[pl.ds(i * _RB, _RB), :]
        y_blk = y_ref[pl.ds(i * _RB, _RB), :]
        o_ref[...] = d_blk * z + d_blk * y_blk + b_ref[...]


def kernel(x, A, W, b):
    n, din = x.shape
    dout = W.shape[0]
    nstripes = n // _RB

    out = pl.pallas_call(
        _gcn_body,
        grid=(2 * nstripes,),
        in_specs=[
            pl.BlockSpec((_RB, n), lambda k: (jnp.minimum(k, nstripes - 1), 0)),
            pl.BlockSpec((n, din), lambda k: (0, 0)),
            pl.BlockSpec((dout, din), lambda k: (0, 0)),
            pl.BlockSpec((1, dout), lambda k: (0, 0)),
        ],
        out_specs=pl.BlockSpec(
            (_RB, dout), lambda k: (jnp.maximum(k - nstripes, 0), 0)
        ),
        out_shape=jax.ShapeDtypeStruct((n, dout), jnp.float32),
        scratch_shapes=[
            pltpu.VMEM((nstripes, _RB, n), jnp.int8),
            pltpu.VMEM((n, 1), jnp.float32),
            pltpu.VMEM((n, dout), jnp.float32),
            pltpu.VMEM((n, dout), jnp.bfloat16),
        ],
    )(A, x, W, b.reshape(1, dout))
    return out
